# baseline (speedup 1.0000x reference)
import functools

import jax
import jax.numpy as jnp
from jax import lax
from jax.experimental import pallas as pl
from jax.experimental.pallas import tpu as pltpu
from jax.experimental.pallas import tpu_sc as plsc

BATCH = 16384
N_FIELDS = 26
EMBEDDING_DIM = 64
NUM_EMB = 1000000

_B = BATCH * N_FIELDS
_NC = 2
_NS = 16
_NW = _NC * _NS
_NSLAB = NUM_EMB // 128
_SLABS_PER_W = -(-_NSLAB // _NW)
_TAIL_COL = _NSLAB * 128

_B_PER_W = BATCH // _NW
_BBLK = 32
_NBLK = _B_PER_W // _BBLK
_BLK_ROWS = _BBLK * N_FIELDS
_CHUNK = 104
_N_CHUNKS = _BLK_ROWS // _CHUNK

_mesh = plsc.VectorSubcoreMesh(core_axis_name="c", subcore_axis_name="s")


@functools.partial(
    pl.kernel,
    mesh=_mesh,
    out_type=jax.ShapeDtypeStruct((NUM_EMB * 128,), jnp.float32),
    scratch_types=[
        pltpu.VMEM((EMBEDDING_DIM, 128), jnp.float32),
        pltpu.VMEM((EMBEDDING_DIM, 128), jnp.float32),
        pltpu.VMEM((128 * 128,), jnp.float32),
        pltpu.VMEM((128 * 128,), jnp.float32),
        pltpu.VMEM((EMBEDDING_DIM, 64), jnp.float32),
        pltpu.VMEM((64 * 128,), jnp.float32),
        pltpu.SemaphoreType.DMA((2,)),
        pltpu.SemaphoreType.DMA((2,)),
    ],
    compiler_params=pltpu.CompilerParams(needs_layout_passes=False),
)
def _sc_format(wt_hbm, out_hbm, slab_v0, slab_v1, stage_v0, stage_v1,
               tslab_v, tstage_v, lsems, ssems):
    slabs = (slab_v0, slab_v1)
    stages = (stage_v0, stage_v1)
    wid = lax.axis_index("s") * _NC + lax.axis_index("c")
    d16 = lax.iota(jnp.int32, 16)
    d16_128 = d16 * 128

    def slab_col(k):
        s = wid + k * _NW
        return s, pl.multiple_of(s * 128, 128)

    def start_load(k, rb):
        s, col = slab_col(k)

        @pl.when(s < _NSLAB)
        def _():
            pltpu.async_copy(
                wt_hbm.at[:, pl.ds(col, 128)], slabs[rb], lsems.at[rb]
            )

    def wait_load(rb):
        pltpu.make_async_copy(
            wt_hbm.at[:, pl.ds(0, 128)], slabs[rb], lsems.at[rb]
        ).wait()

    def wait_store(rb):
        pltpu.make_async_copy(
            stages[rb], out_hbm.at[pl.ds(0, 128 * 128)], ssems.at[rb]
        ).wait()

    def do_slab(k, rb, first):
        s, col = slab_col(k)

        @pl.when(s < _NSLAB)
        def _():
            wait_load(rb)
            if not first:
                wait_store(rb)
            for d in range(EMBEDDING_DIM):
                for j in range(8):
                    val = slabs[rb][d, pl.ds(j * 16, 16)]
                    plsc.store_scatter(
                        stages[rb], [d16_128 + (j * 16 * 128 + d)], val
                    )
            start_load(k + 2, rb)
            pltpu.async_copy(
                stages[rb], out_hbm.at[pl.ds(col * 128, 128 * 128)],
                ssems.at[rb],
            )

    start_load(0, 0)
    start_load(1, 1)

    def pair_body(kp, carry):
        do_slab(kp * 2, 0, False)
        do_slab(kp * 2 + 1, 1, False)
        return carry

    do_slab(0, 0, True)
    do_slab(1, 1, True)
    lax.fori_loop(1, -(-_SLABS_PER_W // 2), pair_body, 0)
    wait_store(0)
    wait_store(1)

    @pl.when(wid == _NSLAB % _NW)
    def _tail():
        pltpu.sync_copy(wt_hbm.at[:, pl.ds(_TAIL_COL, 64)], tslab_v)
        for d in range(EMBEDDING_DIM):
            for j in range(4):
                val = tslab_v[d, pl.ds(j * 16, 16)]
                plsc.store_scatter(
                    tstage_v, [d16_128 + (j * 16 * 128 + d)], val
                )
        pltpu.sync_copy(
            tstage_v, out_hbm.at[pl.ds(_TAIL_COL * 128, 64 * 128)]
        )


@functools.partial(
    pl.kernel,
    mesh=_mesh,
    out_type=jax.ShapeDtypeStruct((N_FIELDS, EMBEDDING_DIM, BATCH), jnp.float32),
    scratch_types=[
        pltpu.VMEM((_BLK_ROWS,), jnp.int32),
        pltpu.VMEM((2, _CHUNK, 128), jnp.float32),
        pltpu.VMEM((N_FIELDS, EMBEDDING_DIM, _BBLK), jnp.float32),
        pltpu.SemaphoreType.DMA((2,)),
        pltpu.SemaphoreType.DMA,
        pltpu.SemaphoreType.DMA,
    ],
    compiler_params=pltpu.CompilerParams(
        use_tc_tiling_on_sc=False, needs_layout_passes=False
    ),
)
def _sc_gather_t(idx_hbm, table_hbm, out_hbm, idx_v, rows_v, blk_v, gsems,
                 isem, bsem):
    wid = lax.axis_index("s") * _NC + lax.axis_index("c")
    d16 = lax.iota(jnp.int32, 16)
    d_vecs = [d16 + (j * 16) for j in range(4)]
    f_vecs = [jnp.full((16,), f, jnp.int32) for f in range(N_FIELDS)]

    def start_gather(c, rb):
        pltpu.async_copy(
            table_hbm.at[idx_v.at[pl.ds(c * _CHUNK, _CHUNK)]],
            rows_v.at[rb],
            gsems.at[rb],
        )

    def wait_gather(rb):
        pltpu.make_async_copy(
            table_hbm.at[idx_v.at[pl.ds(0, _CHUNK)]], rows_v.at[rb],
            gsems.at[rb],
        ).wait()

    def wait_block_store():
        pltpu.make_async_copy(
            blk_v, out_hbm.at[:, :, pl.ds(0, _BBLK)], bsem
        ).wait()

    def scatter_chunk(c, rb):
        b_base = jnp.full((16,), c * 4, jnp.int32)
        b_vecs = [b_base + q for q in range(4)]
        for i in range(_CHUNK):
            f = i % N_FIELDS
            q = i // N_FIELDS
            for j in range(4):
                val = rows_v[rb, i, pl.ds(j * 16, 16)]
                plsc.store_scatter(
                    blk_v, [f_vecs[f], d_vecs[j], b_vecs[q]], val
                )

    def block_body(blk, carry):
        base_b = wid * _B_PER_W + blk * _BBLK
        pltpu.async_copy(
            idx_hbm.at[pl.ds(base_b * N_FIELDS, _BLK_ROWS)], idx_v, isem
        ).wait()
        start_gather(0, 0)

        def chunk_pair(cp, carry2):
            for sub in range(2):
                c = cp * 2 + sub
                rb = sub
                @pl.when(c + 1 < _N_CHUNKS)
                def _():
                    start_gather(c + 1, 1 - rb)
                wait_gather(rb)

                @pl.when(jnp.logical_and(blk > 0, c == 0))
                def _():
                    wait_block_store()

                scatter_chunk(c, rb)
            return carry2

        lax.fori_loop(0, _N_CHUNKS // 2, chunk_pair, 0)
        pltpu.async_copy(blk_v, out_hbm.at[:, :, pl.ds(base_b, _BBLK)], bsem)
        return carry

    lax.fori_loop(0, _NBLK, block_body, 0)
    wait_block_store()


def kernel(token_ids, weight):
    idx_flat = jnp.reshape(token_ids, (_B,)).astype(jnp.int32)
    wt = jnp.transpose(weight)
    flat = _sc_format(wt)
    table = jnp.reshape(flat, (NUM_EMB, 128))
    out_t = _sc_gather_t(idx_flat, table)
    return jnp.transpose(out_t, (2, 0, 1))


# speedup vs baseline: 1.5267x; 1.5267x over previous
import functools

import jax
import jax.numpy as jnp
from jax import lax
from jax.experimental import pallas as pl
from jax.experimental.pallas import tpu as pltpu
from jax.experimental.pallas import tpu_sc as plsc

BATCH = 16384
N_FIELDS = 26
EMBEDDING_DIM = 64

_B = BATCH * N_FIELDS
_NC = 2
_NS = 16
_NW = _NC * _NS
_B_PER_W = BATCH // _NW
_BCHUNK = 128
_KB = _B_PER_W // _BCHUNK
_N_UNITS = N_FIELDS * _KB

_mesh = plsc.VectorSubcoreMesh(core_axis_name="c", subcore_axis_name="s")


@functools.partial(
    pl.kernel,
    mesh=_mesh,
    out_type=jax.ShapeDtypeStruct((N_FIELDS, EMBEDDING_DIM, BATCH), jnp.float32),
    scratch_types=[
        pltpu.VMEM((_B_PER_W * N_FIELDS,), jnp.int32),
        pltpu.VMEM((2, _BCHUNK), jnp.int32),
        pltpu.VMEM((2, _BCHUNK, EMBEDDING_DIM), jnp.float32),
        pltpu.VMEM((2, EMBEDDING_DIM, _BCHUNK), jnp.float32),
        pltpu.SemaphoreType.DMA((2,)),
        pltpu.SemaphoreType.DMA((2,)),
    ],
    compiler_params=pltpu.CompilerParams(
        use_tc_tiling_on_sc=False, needs_layout_passes=False
    ),
)
def _sc_gather_t(idx_hbm, table_hbm, out_hbm, idx_v, glist_v, rows_v, slab_v,
                 gsems, ssems):
    wid = lax.axis_index("s") * _NC + lax.axis_index("c")
    b_lo = wid * _B_PER_W
    lane = lax.iota(jnp.int32, 16)
    lane26 = lane * N_FIELDS

    pltpu.sync_copy(idx_hbm.at[pl.ds(b_lo * N_FIELDS, _B_PER_W * N_FIELDS)],
                    idx_v)

    def unit_fk(u):
        f = u // _KB
        kb = u - f * _KB
        return f, kb

    def build_glist(u, rb):
        f, kb = unit_fk(u)
        base = (kb * _BCHUNK) * N_FIELDS + f
        for j in range(8):
            pos = lane26 + (base + j * 16 * N_FIELDS)
            vals = plsc.load_gather(idx_v, [pos])
            glist_v[rb, pl.ds(j * 16, 16)] = vals

    def start_gather(rb):
        pltpu.async_copy(
            table_hbm.at[glist_v.at[rb]], rows_v.at[rb], gsems.at[rb]
        )

    def wait_gather(rb):
        pltpu.make_async_copy(
            table_hbm.at[glist_v.at[rb]], rows_v.at[rb], gsems.at[rb]
        ).wait()

    def start_store(u, rb):
        f, kb = unit_fk(u)
        pltpu.async_copy(
            slab_v.at[rb],
            out_hbm.at[f, :, pl.ds(b_lo + kb * _BCHUNK, _BCHUNK)],
            ssems.at[rb],
        )

    def wait_store(rb):
        pltpu.make_async_copy(
            slab_v.at[rb], out_hbm.at[0, :, pl.ds(0, _BCHUNK)], ssems.at[rb]
        ).wait()

    def transpose_chunk(rb):
        for r in range(_BCHUNK):
            r_vec = jnp.full((16,), r, jnp.int32)
            for j in range(EMBEDDING_DIM // 16):
                val = rows_v[rb, r, pl.ds(j * 16, 16)]
                plsc.store_scatter(
                    slab_v.at[rb], [lane + (j * 16), r_vec], val
                )

    build_glist(0, 0)
    start_gather(0)
    build_glist(1, 1)
    start_gather(1)

    def unit_pair(up, carry):
        for sub in range(2):
            u = up * 2 + sub
            rb = sub
            wait_gather(rb)

            @pl.when(u >= 2)
            def _():
                wait_store(rb)

            transpose_chunk(rb)
            start_store(u, rb)

            @pl.when(u + 2 < _N_UNITS)
            def _():
                build_glist(u + 2, rb)
                start_gather(rb)
        return carry

    lax.fori_loop(0, _N_UNITS // 2, unit_pair, 0)
    wait_store(0)
    wait_store(1)


def kernel(token_ids, weight):
    idx_flat = jnp.reshape(token_ids, (_B,)).astype(jnp.int32)
    out_t = _sc_gather_t(idx_flat, weight)
    return jnp.transpose(out_t, (2, 0, 1))


# speedup vs baseline: 1.9294x; 1.2638x over previous
import functools

import jax
import jax.numpy as jnp
from jax import lax
from jax.experimental import pallas as pl
from jax.experimental.pallas import tpu as pltpu
from jax.experimental.pallas import tpu_sc as plsc

BATCH = 16384
N_FIELDS = 26
EMBEDDING_DIM = 64

_B = BATCH * N_FIELDS
_NC = 2
_NS = 16
_NW = _NC * _NS
_B_PER_W = BATCH // _NW
_BCHUNK = 128
_KB = _B_PER_W // _BCHUNK
_N_UNITS = N_FIELDS * _KB

_mesh = plsc.VectorSubcoreMesh(core_axis_name="c", subcore_axis_name="s")


@functools.partial(
    pl.kernel,
    mesh=_mesh,
    out_type=jax.ShapeDtypeStruct((N_FIELDS, EMBEDDING_DIM, BATCH), jnp.float32),
    scratch_types=[
        pltpu.VMEM((_B_PER_W * N_FIELDS,), jnp.int32),
        pltpu.VMEM((2, _BCHUNK), jnp.int32),
        pltpu.VMEM((2, _BCHUNK, EMBEDDING_DIM), jnp.float32),
        pltpu.VMEM((2, EMBEDDING_DIM, _BCHUNK + 1), jnp.float32),
        pltpu.SemaphoreType.DMA((2,)),
        pltpu.SemaphoreType.DMA((2,)),
    ],
    compiler_params=pltpu.CompilerParams(
        use_tc_tiling_on_sc=False, needs_layout_passes=False
    ),
)
def _sc_gather_t(idx_hbm, table_hbm, out_hbm, idx_v, glist_v, rows_v, slab_v,
                 gsems, ssems):
    wid = lax.axis_index("s") * _NC + lax.axis_index("c")
    b_lo = wid * _B_PER_W
    lane = lax.iota(jnp.int32, 16)
    lane26 = lane * N_FIELDS

    pltpu.sync_copy(idx_hbm.at[pl.ds(b_lo * N_FIELDS, _B_PER_W * N_FIELDS)],
                    idx_v)

    def unit_fk(u):
        f = u // _KB
        kb = u - f * _KB
        return f, kb

    def build_glist(u, rb):
        f, kb = unit_fk(u)
        base = (kb * _BCHUNK) * N_FIELDS + f
        for j in range(8):
            pos = lane26 + (base + j * 16 * N_FIELDS)
            vals = plsc.load_gather(idx_v, [pos])
            glist_v[rb, pl.ds(j * 16, 16)] = vals

    def start_gather(rb):
        pltpu.async_copy(
            table_hbm.at[glist_v.at[rb]], rows_v.at[rb], gsems.at[rb]
        )

    def wait_gather(rb):
        pltpu.make_async_copy(
            table_hbm.at[glist_v.at[rb]], rows_v.at[rb], gsems.at[rb]
        ).wait()

    def start_store(u, rb):
        f, kb = unit_fk(u)
        pltpu.async_copy(
            slab_v.at[rb, :, pl.ds(0, _BCHUNK)],
            out_hbm.at[f, :, pl.ds(b_lo + kb * _BCHUNK, _BCHUNK)],
            ssems.at[rb],
        )

    def wait_store(rb):
        pltpu.make_async_copy(
            slab_v.at[rb, :, pl.ds(0, _BCHUNK)],
            out_hbm.at[0, :, pl.ds(0, _BCHUNK)], ssems.at[rb]
        ).wait()

    def transpose_chunk(rb):
        for r in range(_BCHUNK):
            r_vec = jnp.full((16,), r, jnp.int32)
            for j in range(EMBEDDING_DIM // 16):
                val = rows_v[rb, r, pl.ds(j * 16, 16)]
                plsc.store_scatter(
                    slab_v.at[rb], [lane + (j * 16), r_vec], val
                )

    build_glist(0, 0)
    start_gather(0)
    build_glist(1, 1)
    start_gather(1)

    def unit_pair(up, carry):
        for sub in range(2):
            u = up * 2 + sub
            rb = sub
            wait_gather(rb)

            @pl.when(u >= 2)
            def _():
                wait_store(rb)

            transpose_chunk(rb)
            start_store(u, rb)

            @pl.when(u + 2 < _N_UNITS)
            def _():
                build_glist(u + 2, rb)
                start_gather(rb)
        return carry

    lax.fori_loop(0, _N_UNITS // 2, unit_pair, 0)
    wait_store(0)
    wait_store(1)


def kernel(token_ids, weight):
    idx_flat = jnp.reshape(token_ids, (_B,)).astype(jnp.int32)
    out_t = _sc_gather_t(idx_flat, weight)
    return jnp.transpose(out_t, (2, 0, 1))
